# Initial kernel scaffold; baseline (speedup 1.0000x reference)
import functools

import jax
import jax.numpy as jnp
from jax import lax
from jax.experimental import pallas as pl
from jax.experimental.pallas import tpu as pltpu
from jax.experimental.pallas import tpu_sc as plsc

N_NODES = 10000
N_EDGES = 320000
D = 128

NC = 2
NS = 16
NW = NC * NS
EPW = N_EDGES // NW
CHUNK = 80
NCHUNK = EPW // CHUNK
ROWS_PER_SUB = N_NODES // NS
ZROWS = 125


def _prescale_body(h_ref, n_ref, o_ref):
    o_ref[...] = h_ref[...] * n_ref[...]


def _finish_body(p_ref, w_ref, n_ref, o_ref):
    agg = p_ref[0] + p_ref[1]
    mm = jnp.dot(agg, w_ref[...], preferred_element_type=jnp.float32)
    o_ref[...] = jnp.maximum(mm * n_ref[...], 0.0)


def _agg_body(hn_hbm, src_hbm, dst_hbm, out_hbm,
              src_v, dst_v, rows_v, zbuf, acc, sem):
    c = lax.axis_index("c")
    s = lax.axis_index("s")
    wid = c * NS + s

    def zstore(i, carry):
        zbuf[i // 8, pl.ds((i % 8) * 16, 16)] = jnp.zeros((16,), jnp.float32)
        return carry
    lax.fori_loop(0, ZROWS * 8, zstore, 0)

    def zcopy(k, carry):
        pltpu.sync_copy(zbuf, acc.at[pl.ds(s * ROWS_PER_SUB + k * ZROWS, ZROWS)])
        return carry
    lax.fori_loop(0, ROWS_PER_SUB // ZROWS, zcopy, 0)
    plsc.subcore_barrier()

    pltpu.sync_copy(src_hbm.at[pl.ds(wid * EPW, EPW)], src_v)

    def body(k, carry):
        base = k * CHUNK
        pltpu.sync_copy(dst_hbm.at[pl.ds(wid * EPW + base, CHUNK)], dst_v)
        pltpu.async_copy(hn_hbm.at[src_v.at[pl.ds(base, CHUNK)]], rows_v,
                         sem).wait()
        pltpu.sync_copy(rows_v, acc.at[dst_v], add=True)
        return carry
    lax.fori_loop(0, NCHUNK, body, 0)

    plsc.subcore_barrier()
    pltpu.sync_copy(acc.at[pl.ds(s * ROWS_PER_SUB, ROWS_PER_SUB)],
                    out_hbm.at[c, pl.ds(s * ROWS_PER_SUB, ROWS_PER_SUB)])


_agg = functools.partial(
    pl.kernel,
    mesh=plsc.VectorSubcoreMesh(core_axis_name="c", subcore_axis_name="s"),
    out_type=jax.ShapeDtypeStruct((NC, N_NODES, D), jnp.float32),
    scratch_types=[
        pltpu.VMEM((EPW,), jnp.int32),
        pltpu.VMEM((CHUNK,), jnp.int32),
        pltpu.VMEM((CHUNK, D), jnp.float32),
        pltpu.VMEM((ZROWS, D), jnp.float32),
        pltpu.VMEM_SHARED((N_NODES, D), jnp.float32),
        pltpu.SemaphoreType.DMA,
    ],
)(_agg_body)


def kernel(h, edge_index, W, norm):
    src = edge_index[0].astype(jnp.int32)
    dst = edge_index[1].astype(jnp.int32)

    hn = pl.pallas_call(
        _prescale_body,
        out_shape=jax.ShapeDtypeStruct((N_NODES, D), jnp.float32),
    )(h, norm)

    partials = _agg(hn, src, dst)

    out = pl.pallas_call(
        _finish_body,
        out_shape=jax.ShapeDtypeStruct((N_NODES, D), jnp.float32),
    )(partials, W, norm)
    return out


# speedup vs baseline: 6.3453x; 6.3453x over previous
import functools

import jax
import jax.numpy as jnp
from jax import lax
from jax.experimental import pallas as pl
from jax.experimental.pallas import tpu as pltpu
from jax.experimental.pallas import tpu_sc as plsc

N_NODES = 10000
N_EDGES = 320000
D = 128

NC = 2
NS = 16
NW = NC * NS
EPW = N_EDGES // NW
CHUNK = 80
NCHUNK = EPW // CHUNK
SUB_ROWS = 624
TAIL_BASE = NS * SUB_ROWS
TAIL_ROWS = N_NODES - TAIL_BASE
ZROWS = 16


def _prescale_body(h_ref, n_ref, o_ref):
    o_ref[...] = h_ref[...] * n_ref[...]


def _finish_body(p_ref, w_ref, n_ref, o_ref):
    agg = p_ref[0] + p_ref[1]
    mm = jnp.dot(agg, w_ref[...], preferred_element_type=jnp.float32)
    o_ref[...] = jnp.maximum(mm * n_ref[...], 0.0)


def _agg_body(hn_hbm, src_hbm, dst_hbm, out_hbm,
              src_v, dst_v, rows_v, zbuf, acc, sem):
    c = lax.axis_index("c")
    s = lax.axis_index("s")
    wid = c * NS + s

    def zstore(i, carry):
        zbuf[i // 8, pl.ds((i % 8) * 16, 16)] = jnp.zeros((16,), jnp.float32)
        return carry
    lax.fori_loop(0, ZROWS * 8, zstore, 0)

    def zcopy(k, carry):
        pltpu.sync_copy(zbuf, acc.at[pl.ds(s * SUB_ROWS + k * ZROWS, ZROWS)])
        return carry
    lax.fori_loop(0, SUB_ROWS // ZROWS, zcopy, 0)

    @pl.when(s == NS - 1)
    def _():
        pltpu.sync_copy(zbuf, acc.at[pl.ds(TAIL_BASE, TAIL_ROWS)])
    plsc.subcore_barrier()

    pltpu.sync_copy(src_hbm.at[pl.ds(wid * EPW, EPW)], src_v)

    def body(k, carry):
        base = k * CHUNK
        pltpu.sync_copy(dst_hbm.at[pl.ds(wid * EPW + base, CHUNK)], dst_v)
        pltpu.async_copy(hn_hbm.at[src_v.at[pl.ds(base, CHUNK)]], rows_v,
                         sem).wait()
        pltpu.sync_copy(rows_v, acc.at[dst_v], add=True)
        return carry
    lax.fori_loop(0, NCHUNK, body, 0)

    plsc.subcore_barrier()
    pltpu.sync_copy(acc.at[pl.ds(s * SUB_ROWS, SUB_ROWS)],
                    out_hbm.at[c, pl.ds(s * SUB_ROWS, SUB_ROWS)])

    @pl.when(s == NS - 1)
    def _():
        pltpu.sync_copy(acc.at[pl.ds(TAIL_BASE, TAIL_ROWS)],
                        out_hbm.at[c, pl.ds(TAIL_BASE, TAIL_ROWS)])


_agg = functools.partial(
    pl.kernel,
    mesh=plsc.VectorSubcoreMesh(core_axis_name="c", subcore_axis_name="s"),
    out_type=jax.ShapeDtypeStruct((NC, N_NODES, D), jnp.float32),
    scratch_types=[
        pltpu.VMEM((EPW,), jnp.int32),
        pltpu.VMEM((CHUNK,), jnp.int32),
        pltpu.VMEM((CHUNK, D), jnp.float32),
        pltpu.VMEM((ZROWS, D), jnp.float32),
        pltpu.VMEM_SHARED((N_NODES, D), jnp.float32),
        pltpu.SemaphoreType.DMA,
    ],
)(_agg_body)


def kernel(h, edge_index, W, norm):
    src = edge_index[0].astype(jnp.int32)
    dst = edge_index[1].astype(jnp.int32)

    hn = pl.pallas_call(
        _prescale_body,
        out_shape=jax.ShapeDtypeStruct((N_NODES, D), jnp.float32),
    )(h, norm)

    partials = _agg(hn, src, dst)

    out = pl.pallas_call(
        _finish_body,
        out_shape=jax.ShapeDtypeStruct((N_NODES, D), jnp.float32),
    )(partials, W, norm)
    return out
